# baseline (speedup 1.0000x reference)
import jax
import jax.numpy as jnp
from jax import lax
from jax.experimental import pallas as pl
from jax.experimental.pallas import tpu as pltpu
from jax.experimental.pallas import tpu_sc as plsc

_B = 4096
_L = 200
_D = 64
_N = _B * _L
_NC = 2
_NS = 16
_NW = _NC * _NS
_PER_W = _N // _NW
_CHUNK = 128
_NCH = _PER_W // _CHUNK
_NBUF = 2


def _gather_body(idx_h, table_h, out_h, idx_v, rows0, rows1, sem0, sem1):
    wid = lax.axis_index("s") * _NC + lax.axis_index("c")
    pltpu.sync_copy(idx_h.at[wid], idx_v)
    rows = (rows0, rows1)
    sems = (sem0, sem1)
    for b in range(_NBUF):
        pltpu.async_copy(table_h.at[idx_v.at[b]], rows[b], sems[b])

    def body(i, _):
        c = i * _NBUF
        for b in range(_NBUF):
            g = c + b
            pltpu.make_async_copy(table_h.at[idx_v.at[0]], rows[b], sems[b]).wait()
            pltpu.sync_copy(rows[b], out_h.at[wid, pl.ds(g * _CHUNK, _CHUNK)])
            nxt = g + _NBUF

            @pl.when(nxt < _NCH)
            def _():
                pltpu.async_copy(table_h.at[idx_v.at[nxt]], rows[b], sems[b])
        return ()

    lax.fori_loop(0, _NCH // _NBUF, body, (), unroll=False)


def _mm_body(c_ref, w_ref, o_ref):
    o_ref[...] = lax.dot_general(
        c_ref[...], w_ref[...], (((1,), (1,)), ((), ())),
        preferred_element_type=jnp.float32)


def kernel(x, condition, table, W_cond):
    idx = x.reshape(_NW, _NCH, _CHUNK).astype(jnp.int32)

    gather = pl.kernel(
        _gather_body,
        out_type=jax.ShapeDtypeStruct((_NW, _PER_W, _D), jnp.float32),
        mesh=plsc.VectorSubcoreMesh(core_axis_name="c", subcore_axis_name="s"),
        scratch_types=[
            pltpu.VMEM((_NCH, _CHUNK), jnp.int32),
            pltpu.VMEM((_CHUNK, _D), jnp.float32),
            pltpu.VMEM((_CHUNK, _D), jnp.float32),
            pltpu.SemaphoreType.DMA,
            pltpu.SemaphoreType.DMA,
        ],
        compiler_params=pltpu.CompilerParams(use_tc_tiling_on_sc=False),
    )
    x_emb = gather(idx, table).reshape(_B, _L, _D)

    cond_emb = pl.pallas_call(
        _mm_body,
        out_shape=jax.ShapeDtypeStruct((_B, _D), jnp.float32),
    )(condition, W_cond)

    return (x_emb, cond_emb)


# speedup vs baseline: 1.0126x; 1.0126x over previous
import jax
import jax.numpy as jnp
from jax import lax
from jax.experimental import pallas as pl
from jax.experimental.pallas import tpu as pltpu
from jax.experimental.pallas import tpu_sc as plsc

_B = 4096
_L = 200
_D = 64
_N = _B * _L
_NC = 2
_NS = 16
_NW = _NC * _NS
_PER_W = _N // _NW
_CHUNK = 128
_NCH = _PER_W // _CHUNK
_NBUF = 4
_K = 2


def _gather_body(idx_h, table_h, out_h, idx_v, rows, gsems, wsems):
    wid = lax.axis_index("s") * _NC + lax.axis_index("c")
    pltpu.sync_copy(idx_h.at[wid], idx_v)
    for b in range(_K):
        pltpu.async_copy(table_h.at[idx_v.at[b]], rows[b], gsems[b])

    def body(i, _):
        for b in range(_NBUF):
            g = i * _NBUF + b
            pltpu.make_async_copy(table_h.at[idx_v.at[0]], rows[b],
                                  gsems[b]).wait()
            pltpu.async_copy(rows[b], out_h.at[wid, pl.ds(g * _CHUNK, _CHUNK)],
                             wsems[b])
            gp = g + _K
            bp = (b + _K) % _NBUF

            @pl.when(gp >= _NBUF)
            def _():
                pltpu.make_async_copy(
                    rows[bp], out_h.at[wid, pl.ds(0, _CHUNK)], wsems[bp]).wait()

            @pl.when(gp < _NCH)
            def _():
                pltpu.async_copy(table_h.at[idx_v.at[gp]], rows[bp], gsems[bp])
        return ()

    lax.fori_loop(0, _NCH // _NBUF, body, (), unroll=False)
    for b in range(_NBUF - _K, _NBUF):
        pltpu.make_async_copy(rows[b], out_h.at[wid, pl.ds(0, _CHUNK)],
                              wsems[b]).wait()


def _gather_entry(idx_h, table_h, out_h, idx_v,
                  r0, r1, r2, r3, g0, g1, g2, g3, w0, w1, w2, w3):
    _gather_body(idx_h, table_h, out_h, idx_v,
                 (r0, r1, r2, r3), (g0, g1, g2, g3), (w0, w1, w2, w3))


def _mm_body(c_ref, w_ref, o_ref):
    o_ref[...] = lax.dot_general(
        c_ref[...], w_ref[...], (((1,), (1,)), ((), ())),
        preferred_element_type=jnp.float32)


def kernel(x, condition, table, W_cond):
    idx = x.reshape(_NW, _NCH, _CHUNK).astype(jnp.int32)

    gather = pl.kernel(
        _gather_entry,
        out_type=jax.ShapeDtypeStruct((_NW, _PER_W, _D), jnp.float32),
        mesh=plsc.VectorSubcoreMesh(core_axis_name="c", subcore_axis_name="s"),
        scratch_types=(
            [pltpu.VMEM((_NCH, _CHUNK), jnp.int32)]
            + [pltpu.VMEM((_CHUNK, _D), jnp.float32)] * _NBUF
            + [pltpu.SemaphoreType.DMA] * (2 * _NBUF)
        ),
        compiler_params=pltpu.CompilerParams(use_tc_tiling_on_sc=False),
    )
    x_emb = gather(idx, table).reshape(_B, _L, _D)

    cond_emb = pl.pallas_call(
        _mm_body,
        out_shape=jax.ShapeDtypeStruct((_B, _D), jnp.float32),
    )(condition, W_cond)

    return (x_emb, cond_emb)


# speedup vs baseline: 1.0157x; 1.0030x over previous
import jax
import jax.numpy as jnp
from jax import lax
from jax.experimental import pallas as pl
from jax.experimental.pallas import tpu as pltpu
from jax.experimental.pallas import tpu_sc as plsc

_B = 4096
_L = 200
_D = 64
_N = _B * _L
_NC = 2
_NS = 16
_NW = _NC * _NS
_PER_W = _N // _NW
_CHUNK = 128
_NCH = _PER_W // _CHUNK
_NBUF = 4
_K = 2


def _gather_body(idx_h, table_h, out_h, idx_v, rows, gsems, wsems):
    wid = lax.axis_index("s") * _NC + lax.axis_index("c")
    pltpu.sync_copy(idx_h.at[wid], idx_v)
    for b in range(_K):
        pltpu.async_copy(table_h.at[idx_v.at[b]], rows[b], gsems[b])

    def body(i, _):
        for b in range(_NBUF):
            g = i * _NBUF + b
            pltpu.make_async_copy(table_h.at[idx_v.at[0]], rows[b],
                                  gsems[b]).wait()
            pltpu.async_copy(rows[b], out_h.at[wid, pl.ds(g * _CHUNK, _CHUNK)],
                             wsems[b])
            gp = g + _K
            bp = (b + _K) % _NBUF

            @pl.when(gp >= _NBUF)
            def _():
                pltpu.make_async_copy(
                    rows[bp], out_h.at[wid, pl.ds(0, _CHUNK)], wsems[bp]).wait()

            @pl.when(gp < _NCH)
            def _():
                pltpu.async_copy(table_h.at[idx_v.at[gp]], rows[bp], gsems[bp])
        return ()

    lax.fori_loop(0, _NCH // _NBUF, body, (), unroll=False)
    for b in range(_NBUF - _K, _NBUF):
        pltpu.make_async_copy(rows[b], out_h.at[wid, pl.ds(0, _CHUNK)],
                              wsems[b]).wait()


def _gather_entry(idx_h, table_h, out_h, idx_v,
                  r0, r1, r2, r3, g0, g1, g2, g3, w0, w1, w2, w3):
    _gather_body(idx_h, table_h, out_h, idx_v,
                 (r0, r1, r2, r3), (g0, g1, g2, g3), (w0, w1, w2, w3))


def _mm_body(c_ref, w_ref, o_ref):
    o_ref[...] = lax.dot_general(
        c_ref[...], w_ref[...], (((1,), (1,)), ((), ())),
        preferred_element_type=jnp.float32)


def kernel(x, condition, table, W_cond):
    idx = x.reshape(_NW, _NCH, _CHUNK).astype(jnp.int32)

    gather = pl.kernel(
        _gather_entry,
        out_type=jax.ShapeDtypeStruct((_NW, _PER_W, _D), jnp.float32),
        mesh=plsc.VectorSubcoreMesh(core_axis_name="c", subcore_axis_name="s"),
        scratch_types=(
            [pltpu.VMEM((_NCH, _CHUNK), jnp.int32)]
            + [pltpu.VMEM((_CHUNK, _D), jnp.float32)] * _NBUF
            + [pltpu.SemaphoreType.DMA] * (2 * _NBUF)
        ),
        compiler_params=pltpu.CompilerParams(use_tc_tiling_on_sc=False,
                                             skip_device_barrier=True),
    )
    x_emb = gather(idx, table).reshape(_B, _L, _D)

    cond_emb = pl.pallas_call(
        _mm_body,
        out_shape=jax.ShapeDtypeStruct((_B, _D), jnp.float32),
    )(condition, W_cond)

    return (x_emb, cond_emb)
